# baseline (speedup 1.0000x reference)
import functools

import jax
import jax.numpy as jnp
from jax import lax
from jax.experimental import pallas as pl
from jax.experimental.pallas import tpu as pltpu
from jax.experimental.pallas import tpu_sc as plsc

B, L, D = 16384, 50, 64
R = B * L
NC, NS = 2, 16
NW = NC * NS
ROWS_PER_W = R // NW
CHUNK = 512
SUB = 128
NSUB = CHUNK // SUB
N_CHUNKS = ROWS_PER_W // CHUNK
ROUNDS = N_CHUNKS // 2

_MESH = plsc.VectorSubcoreMesh(core_axis_name="c", subcore_axis_name="s",
                               num_cores=NC, num_subcores=NS)


_DNUMS = lax.GatherDimensionNumbers(
    offset_dims=(), collapsed_slice_dims=(0,), start_index_map=(0,))


def _lane_shuffle(x, idx):
    return lax.gather(x, idx[:, None], _DNUMS, (1,),
                      mode=lax.GatherScatterMode.PROMISE_IN_BOUNDS)


@functools.partial(
    pl.kernel,
    out_type=jax.ShapeDtypeStruct((R, D), jnp.float32),
    mesh=_MESH,
    compiler_params=pltpu.CompilerParams(use_tc_tiling_on_sc=False),
    scratch_types=[
        pltpu.VMEM((NSUB, SUB), jnp.int32),
        pltpu.VMEM((NSUB, SUB), jnp.int32),
        pltpu.VMEM((CHUNK, D), jnp.float32),
        pltpu.VMEM((CHUNK, D), jnp.float32),
        pltpu.SemaphoreType.DMA,
        pltpu.SemaphoreType.DMA,
        pltpu.SemaphoreType.DMA,
        pltpu.SemaphoreType.DMA,
    ],
)
def _gather_norm(idx_hbm, table_hbm, out_hbm,
                 idx0_v, idx1_v, rows0_v, rows1_v,
                 gsem0, gsem1, wsem0, wsem1):
    wid = lax.axis_index("s") * NC + lax.axis_index("c")
    base = wid * ROWS_PER_W
    lane = lax.iota(jnp.int32, 16)
    perms = [lane ^ sh for sh in (8, 4, 2, 1)]

    bufs = ((idx0_v, rows0_v, gsem0, wsem0),
            (idx1_v, rows1_v, gsem1, wsem1))

    def gather_descs(b):
        idx_v, rows_v, gsem, _ = bufs[b]
        return [pltpu.make_async_copy(table_hbm.at[idx_v.at[s]],
                                      rows_v.at[pl.ds(s * SUB, SUB)], gsem)
                for s in range(NSUB)]

    def issue_gather(c, b):
        idx_v = bufs[b][0]
        i0 = pl.multiple_of((base + c * CHUNK) // SUB, NSUB)
        pltpu.sync_copy(idx_hbm.at[pl.ds(i0, NSUB)], idx_v)
        for d in gather_descs(b):
            d.start()

    def wait_gather(b):
        for d in gather_descs(b):
            d.wait()

    def wb_desc(c, b):
        _, rows_v, _, wsem = bufs[b]
        row0 = pl.multiple_of(base + c * CHUNK, CHUNK)
        return pltpu.make_async_copy(rows_v,
                                     out_hbm.at[pl.ds(row0, CHUNK)], wsem)

    def compute(b):
        rows_v = bufs[b][1]

        @plsc.parallel_loop(0, CHUNK, unroll=16)
        def _row(i):
            v0 = rows_v[i, pl.ds(0, 16)]
            v1 = rows_v[i, pl.ds(16, 16)]
            v2 = rows_v[i, pl.ds(32, 16)]
            v3 = rows_v[i, pl.ds(48, 16)]
            ss = v0 * v0 + v1 * v1 + v2 * v2 + v3 * v3
            for p in perms:
                ss = ss + _lane_shuffle(ss, p)
            seed = jnp.int32(0x5F3759DF) - jnp.right_shift(
                lax.bitcast_convert_type(ss, jnp.int32), 1)
            y = lax.bitcast_convert_type(seed, jnp.float32)
            h = ss * 0.5
            y = y * (1.5 - h * y * y)
            y = y * (1.5 - h * y * y)
            y = y * (1.5 - h * y * y)
            scale = jnp.where(ss > 1.0, y, jnp.float32(1.0))
            rows_v[i, pl.ds(0, 16)] = v0 * scale
            rows_v[i, pl.ds(16, 16)] = v1 * scale
            rows_v[i, pl.ds(32, 16)] = v2 * scale
            rows_v[i, pl.ds(48, 16)] = v3 * scale

    issue_gather(0, 0)

    @pl.loop(0, ROUNDS)
    def _round(r):
        ca = 2 * r
        cb = 2 * r + 1

        wait_gather(0)

        @pl.when(r > 0)
        def _():
            wb_desc(ca - 1, 1).wait()

        issue_gather(cb, 1)
        compute(0)
        wb_desc(ca, 0).start()
        wait_gather(1)
        wb_desc(ca, 0).wait()

        @pl.when(r < ROUNDS - 1)
        def _():
            issue_gather(ca + 2, 0)

        compute(1)
        wb_desc(cb, 1).start()

    wb_desc(N_CHUNKS - 1, 1).wait()


def kernel(indices, table):
    idx = indices.reshape(R // SUB, SUB).astype(jnp.int32)
    out = _gather_norm(idx, table)
    return out.reshape(B, L, D)


# speedup vs baseline: 1.0692x; 1.0692x over previous
import functools

import jax
import jax.numpy as jnp
from jax import lax
from jax.experimental import pallas as pl
from jax.experimental.pallas import tpu as pltpu
from jax.experimental.pallas import tpu_sc as plsc

B, L, D = 16384, 50, 64
R = B * L
NC, NS = 2, 16
NW = NC * NS
ROWS_PER_W = R // NW
CHUNK = 512
SUB = 128
NSUB = CHUNK // SUB
N_CHUNKS = ROWS_PER_W // CHUNK
ROUNDS = N_CHUNKS // 2

_MESH = plsc.VectorSubcoreMesh(core_axis_name="c", subcore_axis_name="s",
                               num_cores=NC, num_subcores=NS)


_DNUMS = lax.GatherDimensionNumbers(
    offset_dims=(), collapsed_slice_dims=(0,), start_index_map=(0,))


def _lane_shuffle(x, idx):
    return lax.gather(x, idx[:, None], _DNUMS, (1,),
                      mode=lax.GatherScatterMode.PROMISE_IN_BOUNDS)


@functools.partial(
    pl.kernel,
    out_type=jax.ShapeDtypeStruct((R, D), jnp.float32),
    mesh=_MESH,
    compiler_params=pltpu.CompilerParams(use_tc_tiling_on_sc=False),
    scratch_types=[
        pltpu.VMEM((NSUB, SUB), jnp.int32),
        pltpu.VMEM((NSUB, SUB), jnp.int32),
        pltpu.VMEM((CHUNK, D), jnp.float32),
        pltpu.VMEM((CHUNK, D), jnp.float32),
        pltpu.SemaphoreType.DMA,
        pltpu.SemaphoreType.DMA,
        pltpu.SemaphoreType.DMA,
        pltpu.SemaphoreType.DMA,
    ],
)
def _gather_norm(idx_hbm, table_hbm, out_hbm,
                 idx0_v, idx1_v, rows0_v, rows1_v,
                 gsem0, gsem1, wsem0, wsem1):
    wid = lax.axis_index("s") * NC + lax.axis_index("c")
    base = wid * ROWS_PER_W
    lane = lax.iota(jnp.int32, 16)
    perms = [lane ^ sh for sh in (8, 4, 2, 1)]

    bufs = ((idx0_v, rows0_v, gsem0, wsem0),
            (idx1_v, rows1_v, gsem1, wsem1))

    def gather_descs(b):
        idx_v, rows_v, gsem, _ = bufs[b]
        return [pltpu.make_async_copy(table_hbm.at[idx_v.at[s]],
                                      rows_v.at[pl.ds(s * SUB, SUB)], gsem)
                for s in range(NSUB)]

    def issue_gather(c, b):
        idx_v = bufs[b][0]
        i0 = pl.multiple_of((base + c * CHUNK) // SUB, NSUB)
        pltpu.sync_copy(idx_hbm.at[pl.ds(i0, NSUB)], idx_v)
        for d in gather_descs(b):
            d.start()

    def wait_gather(b):
        for d in gather_descs(b):
            d.wait()

    def wb_desc(c, b):
        _, rows_v, _, wsem = bufs[b]
        row0 = pl.multiple_of(base + c * CHUNK, CHUNK)
        return pltpu.make_async_copy(rows_v,
                                     out_hbm.at[pl.ds(row0, CHUNK)], wsem)

    def compute(b):
        rows_v = bufs[b][1]

        @plsc.parallel_loop(0, CHUNK, unroll=8)
        def _row(i):
            v0 = rows_v[i, pl.ds(0, 16)]
            v1 = rows_v[i, pl.ds(16, 16)]
            v2 = rows_v[i, pl.ds(32, 16)]
            v3 = rows_v[i, pl.ds(48, 16)]
            ss = v0 * v0 + v1 * v1 + v2 * v2 + v3 * v3
            for p in perms:
                ss = ss + _lane_shuffle(ss, p)
            seed = jnp.int32(0x5F3759DF) - jnp.right_shift(
                lax.bitcast_convert_type(ss, jnp.int32), 1)
            y = lax.bitcast_convert_type(seed, jnp.float32)
            h = ss * 0.5
            y = y * (1.5 - h * y * y)
            y = y * (1.5 - h * y * y)
            scale = jnp.where(ss > 1.0, y, jnp.float32(1.0))
            rows_v[i, pl.ds(0, 16)] = v0 * scale
            rows_v[i, pl.ds(16, 16)] = v1 * scale
            rows_v[i, pl.ds(32, 16)] = v2 * scale
            rows_v[i, pl.ds(48, 16)] = v3 * scale

    issue_gather(0, 0)

    @pl.loop(0, ROUNDS)
    def _round(r):
        ca = 2 * r
        cb = 2 * r + 1

        wait_gather(0)

        @pl.when(r > 0)
        def _():
            wb_desc(ca - 1, 1).wait()

        issue_gather(cb, 1)
        compute(0)
        wb_desc(ca, 0).start()
        wait_gather(1)
        wb_desc(ca, 0).wait()

        @pl.when(r < ROUNDS - 1)
        def _():
            issue_gather(ca + 2, 0)

        compute(1)
        wb_desc(cb, 1).start()

    wb_desc(N_CHUNKS - 1, 1).wait()


def kernel(indices, table):
    idx = indices.reshape(R // SUB, SUB).astype(jnp.int32)
    out = _gather_norm(idx, table)
    return out.reshape(B, L, D)


# speedup vs baseline: 1.1338x; 1.0605x over previous
import functools

import jax
import jax.numpy as jnp
from jax import lax
from jax.experimental import pallas as pl
from jax.experimental.pallas import tpu as pltpu
from jax.experimental.pallas import tpu_sc as plsc

B, L, D = 16384, 50, 64
R = B * L
NC, NS = 2, 16
NW = NC * NS
ROWS_PER_W = R // NW
CHUNK = 512
SUB = 128
NSUB = CHUNK // SUB
N_CHUNKS = ROWS_PER_W // CHUNK
ROUNDS = N_CHUNKS // 2

_MESH = plsc.VectorSubcoreMesh(core_axis_name="c", subcore_axis_name="s",
                               num_cores=NC, num_subcores=NS)


_DNUMS = lax.GatherDimensionNumbers(
    offset_dims=(), collapsed_slice_dims=(0,), start_index_map=(0,))


def _lane_shuffle(x, idx):
    return lax.gather(x, idx[:, None], _DNUMS, (1,),
                      mode=lax.GatherScatterMode.PROMISE_IN_BOUNDS)


@functools.partial(
    pl.kernel,
    out_type=jax.ShapeDtypeStruct((R, D), jnp.float32),
    mesh=_MESH,
    compiler_params=pltpu.CompilerParams(use_tc_tiling_on_sc=False),
    scratch_types=[
        pltpu.VMEM((NSUB, SUB), jnp.int32),
        pltpu.VMEM((NSUB, SUB), jnp.int32),
        pltpu.VMEM((CHUNK, D), jnp.float32),
        pltpu.VMEM((CHUNK, D), jnp.float32),
        pltpu.SemaphoreType.DMA,
        pltpu.SemaphoreType.DMA,
        pltpu.SemaphoreType.DMA,
        pltpu.SemaphoreType.DMA,
    ],
)
def _gather_norm(idx_hbm, table_hbm, out_hbm,
                 idx0_v, idx1_v, rows0_v, rows1_v,
                 gsem0, gsem1, wsem0, wsem1):
    wid = lax.axis_index("s") * NC + lax.axis_index("c")
    base = wid * ROWS_PER_W
    lane = lax.iota(jnp.int32, 16)
    perms = [lane ^ sh for sh in (8, 4, 2, 1)]

    bufs = ((idx0_v, rows0_v, gsem0, wsem0),
            (idx1_v, rows1_v, gsem1, wsem1))

    def gather_descs(b):
        idx_v, rows_v, gsem, _ = bufs[b]
        return [pltpu.make_async_copy(table_hbm.at[idx_v.at[s]],
                                      rows_v.at[pl.ds(s * SUB, SUB)], gsem)
                for s in range(NSUB)]

    def issue_gather(c, b):
        idx_v = bufs[b][0]
        i0 = pl.multiple_of((base + c * CHUNK) // SUB, NSUB)
        pltpu.sync_copy(idx_hbm.at[pl.ds(i0, NSUB)], idx_v)
        for d in gather_descs(b):
            d.start()

    def wait_gather(b):
        for d in gather_descs(b):
            d.wait()

    def wb_desc(c, b):
        _, rows_v, _, wsem = bufs[b]
        row0 = pl.multiple_of(base + c * CHUNK, CHUNK)
        return pltpu.make_async_copy(rows_v,
                                     out_hbm.at[pl.ds(row0, CHUNK)], wsem)

    def compute(b):
        rows_v = bufs[b][1]

        @plsc.parallel_loop(0, CHUNK, unroll=8,
                            carry=jnp.zeros((16,), jnp.float32))
        def _scan(i, mx):
            v0 = rows_v[i, pl.ds(0, 16)]
            v1 = rows_v[i, pl.ds(16, 16)]
            v2 = rows_v[i, pl.ds(32, 16)]
            v3 = rows_v[i, pl.ds(48, 16)]
            ss = v0 * v0 + v1 * v1 + v2 * v2 + v3 * v3
            for p in perms:
                ss = ss + _lane_shuffle(ss, p)
            return jnp.maximum(mx, ss)

        @pl.when(jnp.squeeze(lax.slice(_scan, (0,), (1,))) > 1.0)
        def _rescale():
            @plsc.parallel_loop(0, CHUNK, unroll=8)
            def _row(i):
                v0 = rows_v[i, pl.ds(0, 16)]
                v1 = rows_v[i, pl.ds(16, 16)]
                v2 = rows_v[i, pl.ds(32, 16)]
                v3 = rows_v[i, pl.ds(48, 16)]
                ss = v0 * v0 + v1 * v1 + v2 * v2 + v3 * v3
                for p in perms:
                    ss = ss + _lane_shuffle(ss, p)
                seed = jnp.int32(0x5F3759DF) - jnp.right_shift(
                    lax.bitcast_convert_type(ss, jnp.int32), 1)
                y = lax.bitcast_convert_type(seed, jnp.float32)
                h = ss * 0.5
                y = y * (1.5 - h * y * y)
                y = y * (1.5 - h * y * y)
                scale = jnp.where(ss > 1.0, y, jnp.float32(1.0))
                rows_v[i, pl.ds(0, 16)] = v0 * scale
                rows_v[i, pl.ds(16, 16)] = v1 * scale
                rows_v[i, pl.ds(32, 16)] = v2 * scale
                rows_v[i, pl.ds(48, 16)] = v3 * scale

    issue_gather(0, 0)

    @pl.loop(0, ROUNDS)
    def _round(r):
        ca = 2 * r
        cb = 2 * r + 1

        wait_gather(0)

        @pl.when(r > 0)
        def _():
            wb_desc(ca - 1, 1).wait()

        issue_gather(cb, 1)
        compute(0)
        wb_desc(ca, 0).start()
        wait_gather(1)
        wb_desc(ca, 0).wait()

        @pl.when(r < ROUNDS - 1)
        def _():
            issue_gather(ca + 2, 0)

        compute(1)
        wb_desc(cb, 1).start()

    wb_desc(N_CHUNKS - 1, 1).wait()


def kernel(indices, table):
    idx = indices.reshape(R // SUB, SUB).astype(jnp.int32)
    out = _gather_norm(idx, table)
    return out.reshape(B, L, D)


# speedup vs baseline: 1.1339x; 1.0001x over previous
import functools

import jax
import jax.numpy as jnp
from jax import lax
from jax.experimental import pallas as pl
from jax.experimental.pallas import tpu as pltpu
from jax.experimental.pallas import tpu_sc as plsc

B, L, D = 16384, 50, 64
R = B * L
NC, NS = 2, 16
NW = NC * NS
ROWS_PER_W = R // NW
CHUNK = 512
SUB = 128
NSUB = CHUNK // SUB
N_CHUNKS = ROWS_PER_W // CHUNK
ROUNDS = N_CHUNKS // 2

_MESH = plsc.VectorSubcoreMesh(core_axis_name="c", subcore_axis_name="s",
                               num_cores=NC, num_subcores=NS)


_DNUMS = lax.GatherDimensionNumbers(
    offset_dims=(), collapsed_slice_dims=(0,), start_index_map=(0,))


def _lane_shuffle(x, idx):
    return lax.gather(x, idx[:, None], _DNUMS, (1,),
                      mode=lax.GatherScatterMode.PROMISE_IN_BOUNDS)


@functools.partial(
    pl.kernel,
    out_type=jax.ShapeDtypeStruct((R, D), jnp.float32),
    mesh=_MESH,
    compiler_params=pltpu.CompilerParams(use_tc_tiling_on_sc=False),
    scratch_types=[
        pltpu.VMEM((NSUB, SUB), jnp.int32),
        pltpu.VMEM((NSUB, SUB), jnp.int32),
        pltpu.VMEM((CHUNK, D), jnp.float32),
        pltpu.VMEM((CHUNK, D), jnp.float32),
        pltpu.SemaphoreType.DMA,
        pltpu.SemaphoreType.DMA,
        pltpu.SemaphoreType.DMA,
        pltpu.SemaphoreType.DMA,
    ],
)
def _gather_norm(idx_hbm, table_hbm, out_hbm,
                 idx0_v, idx1_v, rows0_v, rows1_v,
                 gsem0, gsem1, wsem0, wsem1):
    wid = lax.axis_index("s") * NC + lax.axis_index("c")
    base = wid * ROWS_PER_W
    lane = lax.iota(jnp.int32, 16)
    perms = [lane ^ sh for sh in (8, 4, 2, 1)]

    bufs = ((idx0_v, rows0_v, gsem0, wsem0),
            (idx1_v, rows1_v, gsem1, wsem1))

    def gather_descs(b):
        idx_v, rows_v, gsem, _ = bufs[b]
        return [pltpu.make_async_copy(table_hbm.at[idx_v.at[s]],
                                      rows_v.at[pl.ds(s * SUB, SUB)], gsem)
                for s in range(NSUB)]

    def issue_gather(c, b):
        idx_v = bufs[b][0]
        i0 = pl.multiple_of((base + c * CHUNK) // SUB, NSUB)
        pltpu.sync_copy(idx_hbm.at[pl.ds(i0, NSUB)], idx_v)
        for d in gather_descs(b):
            d.start()

    def wait_gather(b):
        for d in gather_descs(b):
            d.wait()

    def wb_desc(c, b):
        _, rows_v, _, wsem = bufs[b]
        row0 = pl.multiple_of(base + c * CHUNK, CHUNK)
        return pltpu.make_async_copy(rows_v,
                                     out_hbm.at[pl.ds(row0, CHUNK)], wsem)

    def compute(b):
        rows_v = bufs[b][1]

        @plsc.parallel_loop(0, CHUNK, unroll=16,
                            carry=jnp.zeros((16,), jnp.float32))
        def _scan(i, mx):
            v0 = rows_v[i, pl.ds(0, 16)]
            v1 = rows_v[i, pl.ds(16, 16)]
            v2 = rows_v[i, pl.ds(32, 16)]
            v3 = rows_v[i, pl.ds(48, 16)]
            ss = v0 * v0 + v1 * v1 + v2 * v2 + v3 * v3
            for p in perms:
                ss = ss + _lane_shuffle(ss, p)
            return jnp.maximum(mx, ss)

        @pl.when(jnp.squeeze(lax.slice(_scan, (0,), (1,))) > 1.0)
        def _rescale():
            @plsc.parallel_loop(0, CHUNK, unroll=8)
            def _row(i):
                v0 = rows_v[i, pl.ds(0, 16)]
                v1 = rows_v[i, pl.ds(16, 16)]
                v2 = rows_v[i, pl.ds(32, 16)]
                v3 = rows_v[i, pl.ds(48, 16)]
                ss = v0 * v0 + v1 * v1 + v2 * v2 + v3 * v3
                for p in perms:
                    ss = ss + _lane_shuffle(ss, p)
                seed = jnp.int32(0x5F3759DF) - jnp.right_shift(
                    lax.bitcast_convert_type(ss, jnp.int32), 1)
                y = lax.bitcast_convert_type(seed, jnp.float32)
                h = ss * 0.5
                y = y * (1.5 - h * y * y)
                y = y * (1.5 - h * y * y)
                scale = jnp.where(ss > 1.0, y, jnp.float32(1.0))
                rows_v[i, pl.ds(0, 16)] = v0 * scale
                rows_v[i, pl.ds(16, 16)] = v1 * scale
                rows_v[i, pl.ds(32, 16)] = v2 * scale
                rows_v[i, pl.ds(48, 16)] = v3 * scale

    issue_gather(0, 0)

    @pl.loop(0, ROUNDS)
    def _round(r):
        ca = 2 * r
        cb = 2 * r + 1

        wait_gather(0)

        @pl.when(r > 0)
        def _():
            wb_desc(ca - 1, 1).wait()

        issue_gather(cb, 1)
        compute(0)
        wb_desc(ca, 0).start()
        wait_gather(1)
        wb_desc(ca, 0).wait()

        @pl.when(r < ROUNDS - 1)
        def _():
            issue_gather(ca + 2, 0)

        compute(1)
        wb_desc(cb, 1).start()

    wb_desc(N_CHUNKS - 1, 1).wait()


def kernel(indices, table):
    idx = indices.reshape(R // SUB, SUB).astype(jnp.int32)
    out = _gather_norm(idx, table)
    return out.reshape(B, L, D)
